# Initial kernel scaffold; baseline (speedup 1.0000x reference)
import jax
import jax.numpy as jnp
from jax.experimental import pallas as pl


def kernel(r_ij, edge_src, W_a, b_a, W_v, W_d):
    raise NotImplementedError("write your pallas kernel here")


# speedup vs baseline: 29.5313x; 29.5313x over previous
import functools

import jax
import jax.numpy as jnp
import numpy as np
from jax import lax
from jax.experimental import pallas as pl
from jax.experimental.pallas import tpu as pltpu
from jax.experimental.pallas import tpu_sc as plsc

R0 = 6.0
N_NODES = 10000
N_EDGES = 640000
N_RAD = 8
DIM_A, DIM_V, DIM_D = 128, 64, 32

D = 128
NW = 32
CHUNK = 128
CH_PER_W = 160
IDXG = 16
NGRP = CH_PER_W // IDXG
EDGES_PW = CHUNK * CH_PER_W
E_PAD = NW * EDGES_PW
N_TILE_ROWS = 632
N_PAD = 16 * N_TILE_ROWS

FB = 1024


def _feat_body(r_ref, phi_ref):
    r = r_ref[...]
    x_sq = jnp.sum(r * r, axis=1, keepdims=True) * (1.0 / R0)
    w = jnp.maximum(1.0 - x_sq, 0.0)
    s = jnp.sqrt(x_sq)
    npi = lax.broadcasted_iota(jnp.int32, (1, N_RAD), 1).astype(jnp.float32) * np.pi
    rad = jnp.cos(npi * s) * w
    rs = r * (17.0 / R0)
    norm = jnp.sqrt(jnp.sum(rs * rs, axis=1, keepdims=True) + 1e-12)
    rv = rs * (jnp.tanh(norm) / norm)
    pieces = [rad]
    for a in range(3):
        pieces.append(rad * rv[:, a:a + 1])
    for a in range(3):
        for b in range(3):
            pieces.append(rad * (rv[:, a:a + 1] * rv[:, b:b + 1]))
    for _ in range(3):
        pieces.append(jnp.zeros_like(rad))
    phi_ref[...] = jnp.concatenate(pieces, axis=1)


def _features(r_pad):
    return pl.pallas_call(
        _feat_body,
        grid=(E_PAD // FB,),
        in_specs=[pl.BlockSpec((FB, 3), lambda i: (i, 0))],
        out_specs=pl.BlockSpec((FB, D), lambda i: (i, 0)),
        out_shape=jax.ShapeDtypeStruct((E_PAD, D), jnp.float32),
    )(r_pad)


def _scatter_body(phi_hbm, idx_hbm, out_hbm, idx_v, buf_a, buf_b, acc_sh):
    c = lax.axis_index("c")
    s = lax.axis_index("s")
    wid = s * 2 + c
    base = s * N_TILE_ROWS

    @pl.loop(0, CHUNK)
    def _zero_rows(i):
        @pl.loop(0, D // 16)
        def _zero_cols(j):
            buf_a[i, pl.ds(j * 16, 16)] = jnp.zeros((16,), jnp.float32)

    for t in range(4):
        pltpu.sync_copy(buf_a, acc_sh.at[pl.ds(base + t * CHUNK, CHUNK)])
    pltpu.sync_copy(buf_a.at[pl.ds(0, N_TILE_ROWS - 4 * CHUNK)],
                    acc_sh.at[pl.ds(base + 4 * CHUNK, N_TILE_ROWS - 4 * CHUNK)])
    plsc.subcore_barrier()

    ebase = wid * EDGES_PW
    for g in range(NGRP):
        pltpu.sync_copy(idx_hbm.at[wid, pl.ds(g * IDXG, IDXG)], idx_v)

        @pl.loop(0, IDXG)
        def _edge_chunk(k):
            pltpu.sync_copy(
                phi_hbm.at[pl.ds(ebase + (g * IDXG + k) * CHUNK, CHUNK)],
                buf_a)
            pltpu.sync_copy(buf_a, acc_sh.at[idx_v.at[k]], add=True)

    plsc.subcore_barrier()

    obase = c * N_PAD + base
    for t in range(4):
        pltpu.sync_copy(acc_sh.at[pl.ds(base + t * CHUNK, CHUNK)], buf_b)
        pltpu.sync_copy(buf_b, out_hbm.at[pl.ds(obase + t * CHUNK, CHUNK)])
    tail = N_TILE_ROWS - 4 * CHUNK
    pltpu.sync_copy(acc_sh.at[pl.ds(base + 4 * CHUNK, tail)],
                    buf_b.at[pl.ds(0, tail)])
    pltpu.sync_copy(buf_b.at[pl.ds(0, tail)],
                    out_hbm.at[pl.ds(obase + 4 * CHUNK, tail)])


@jax.jit
def _scatter_partials(phi, idx3):
    mesh = plsc.VectorSubcoreMesh(core_axis_name="c", subcore_axis_name="s")
    return pl.kernel(
        _scatter_body,
        out_type=jax.ShapeDtypeStruct((2 * N_PAD, D), jnp.float32),
        mesh=mesh,
        scratch_types=[
            pltpu.VMEM((IDXG, CHUNK), jnp.int32),
            pltpu.VMEM((CHUNK, D), jnp.float32),
            pltpu.VMEM((CHUNK, D), jnp.float32),
            pltpu.VMEM_SHARED((N_PAD, D), jnp.float32),
        ],
    )(phi, idx3)


MB = 400


def _mm_body(p_ref, wa_ref, wv_ref, wd_ref, ba_ref, oa_ref, ov_ref, od_ref):
    a = p_ref[0] + p_ref[1]
    oa_ref[...] = (jnp.dot(a, wa_ref[...], preferred_element_type=jnp.float32)
                   + ba_ref[...])
    ov_ref[...] = jnp.dot(a, wv_ref[...], preferred_element_type=jnp.float32)
    od_ref[...] = jnp.dot(a, wd_ref[...], preferred_element_type=jnp.float32)


def _node_matmuls(p3, wa_b, wv_b, wd_b, ba2):
    full = lambda i: (0, 0)
    return pl.pallas_call(
        _mm_body,
        grid=(N_NODES // MB,),
        in_specs=[
            pl.BlockSpec((2, MB, D), lambda i: (0, i, 0)),
            pl.BlockSpec((D, DIM_A), full),
            pl.BlockSpec((D, 3 * DIM_V), full),
            pl.BlockSpec((D, 9 * DIM_D), full),
            pl.BlockSpec((1, DIM_A), full),
        ],
        out_specs=[
            pl.BlockSpec((MB, DIM_A), lambda i: (i, 0)),
            pl.BlockSpec((MB, 3 * DIM_V), lambda i: (i, 0)),
            pl.BlockSpec((MB, 9 * DIM_D), lambda i: (i, 0)),
        ],
        out_shape=[
            jax.ShapeDtypeStruct((N_NODES, DIM_A), jnp.float32),
            jax.ShapeDtypeStruct((N_NODES, 3 * DIM_V), jnp.float32),
            jax.ShapeDtypeStruct((N_NODES, 9 * DIM_D), jnp.float32),
        ],
    )(p3, wa_b, wv_b, wd_b, ba2)


def kernel(r_ij, edge_src, W_a, b_a, W_v, W_d):
    r_pad = jnp.zeros((E_PAD, 3), jnp.float32).at[:N_EDGES].set(r_ij)
    idx_pad = jnp.concatenate([
        edge_src.astype(jnp.int32),
        jnp.full((E_PAD - N_EDGES,), N_NODES, jnp.int32),
    ]).reshape(NW, CH_PER_W, CHUNK)

    phi = _features(r_pad)
    partials = _scatter_partials(phi, idx_pad).reshape(2, N_PAD, D)

    wa_b = jnp.zeros((D, DIM_A), jnp.float32).at[0:N_RAD].set(W_a)
    wv_b = jnp.zeros((D, 3 * DIM_V), jnp.float32)
    wd_b = jnp.zeros((D, 9 * DIM_D), jnp.float32)
    for x in range(3):
        wv_b = wv_b.at[8 * (1 + x):8 * (2 + x),
                       DIM_V * x:DIM_V * (x + 1)].set(W_v)
        for y in range(3):
            t = 4 + 3 * x + y
            g = 3 * x + y
            wd_b = wd_b.at[8 * t:8 * (t + 1),
                           DIM_D * g:DIM_D * (g + 1)].set(W_d)

    oa, ov, od = _node_matmuls(partials, wa_b, wv_b, wd_b,
                               b_a.reshape(1, DIM_A))
    B_a = oa
    B_v = ov.reshape(N_NODES, 3, DIM_V).transpose(0, 2, 1)
    B_d = od.reshape(N_NODES, 3, 3, DIM_D).transpose(0, 3, 1, 2)
    return (B_a, B_v, B_d)


# speedup vs baseline: 31.4733x; 1.0658x over previous
import jax
import jax.numpy as jnp
import numpy as np
from jax import lax
from jax.experimental import pallas as pl
from jax.experimental.pallas import tpu as pltpu
from jax.experimental.pallas import tpu_sc as plsc

R0 = 6.0
N_NODES = 10000
N_EDGES = 640000
N_RAD = 8
DIM_A, DIM_V, DIM_D = 128, 64, 32

D = 128
NW = 32
CHUNK = 128
CH_PER_W = 160
IDXG = 16
NGRP = CH_PER_W // IDXG
EDGES_PW = CHUNK * CH_PER_W
E_PAD = NW * EDGES_PW
N_TILE_ROWS = 632
N_PAD = 16 * N_TILE_ROWS

FB = 2048

_A_SEL = [0, 1, 2, 3, 1, 1, 1, 2, 2, 2, 3, 3, 3, 0, 0, 0]
_B_SEL = [0, 0, 0, 0, 1, 2, 3, 1, 2, 3, 1, 2, 3, 0, 0, 0]


def _feat_body(rx_ref, ry_ref, rz_ref, phi_ref):
    rx, ry, rz = rx_ref[...], ry_ref[...], rz_ref[...]
    r_sq = rx * rx + ry * ry + rz * rz
    x_sq = r_sq * (1.0 / R0)
    w = jnp.maximum(1.0 - x_sq, 0.0)
    s = jnp.sqrt(x_sq)
    cc = 17.0 / R0
    norm = jnp.sqrt(r_sq * (cc * cc) + 1e-12)
    scale = (jnp.tanh(norm) / norm) * cc
    rvx, rvy, rvz = rx * scale, ry * scale, rz * scale

    f = lax.broadcasted_iota(jnp.int32, (1, D), 1)
    c_lane = (f & 7).astype(jnp.float32) * np.pi
    t_lane = f >> 3
    rad = jnp.cos(c_lane * s) * w

    def sel(code_per_t):
        code = jnp.zeros((1, D), jnp.int32)
        for t, v in enumerate(code_per_t):
            if v:
                code = jnp.where(t_lane == t, v, code)
        out = jnp.ones_like(rad)
        out = jnp.where(code == 1, rvx, out)
        out = jnp.where(code == 2, rvy, out)
        out = jnp.where(code == 3, rvz, out)
        return out

    m = sel(_A_SEL) * sel(_B_SEL)
    valid = (t_lane < 13).astype(jnp.float32)
    phi_ref[...] = rad * m * valid


def _features(rx, ry, rz):
    col = pl.BlockSpec((FB, 1), lambda i: (i, 0))
    return pl.pallas_call(
        _feat_body,
        grid=(E_PAD // FB,),
        in_specs=[col, col, col],
        out_specs=pl.BlockSpec((FB, D), lambda i: (i, 0)),
        out_shape=jax.ShapeDtypeStruct((E_PAD, D), jnp.float32),
    )(rx, ry, rz)


def _scatter_body(phi_hbm, idx_hbm, out_hbm, idx_v, buf_a, buf_b, acc_sh):
    c = lax.axis_index("c")
    s = lax.axis_index("s")
    wid = s * 2 + c
    base = s * N_TILE_ROWS

    @pl.loop(0, CHUNK)
    def _zero_rows(i):
        @pl.loop(0, D // 16)
        def _zero_cols(j):
            buf_a[i, pl.ds(j * 16, 16)] = jnp.zeros((16,), jnp.float32)

    for t in range(4):
        pltpu.sync_copy(buf_a, acc_sh.at[pl.ds(base + t * CHUNK, CHUNK)])
    pltpu.sync_copy(buf_a.at[pl.ds(0, N_TILE_ROWS - 4 * CHUNK)],
                    acc_sh.at[pl.ds(base + 4 * CHUNK, N_TILE_ROWS - 4 * CHUNK)])
    plsc.subcore_barrier()

    ebase = wid * EDGES_PW
    for g in range(NGRP):
        pltpu.sync_copy(idx_hbm.at[wid, pl.ds(g * IDXG, IDXG)], idx_v)

        @pl.loop(0, IDXG)
        def _edge_chunk(k):
            pltpu.sync_copy(
                phi_hbm.at[pl.ds(ebase + (g * IDXG + k) * CHUNK, CHUNK)],
                buf_a)
            pltpu.sync_copy(buf_a, acc_sh.at[idx_v.at[k]], add=True)

    plsc.subcore_barrier()

    obase = c * N_PAD + base
    for t in range(4):
        pltpu.sync_copy(acc_sh.at[pl.ds(base + t * CHUNK, CHUNK)], buf_b)
        pltpu.sync_copy(buf_b, out_hbm.at[pl.ds(obase + t * CHUNK, CHUNK)])
    tail = N_TILE_ROWS - 4 * CHUNK
    pltpu.sync_copy(acc_sh.at[pl.ds(base + 4 * CHUNK, tail)],
                    buf_b.at[pl.ds(0, tail)])
    pltpu.sync_copy(buf_b.at[pl.ds(0, tail)],
                    out_hbm.at[pl.ds(obase + 4 * CHUNK, tail)])


def _scatter_partials(phi, idx3):
    mesh = plsc.VectorSubcoreMesh(core_axis_name="c", subcore_axis_name="s")
    return pl.kernel(
        _scatter_body,
        out_type=jax.ShapeDtypeStruct((2 * N_PAD, D), jnp.float32),
        mesh=mesh,
        scratch_types=[
            pltpu.VMEM((IDXG, CHUNK), jnp.int32),
            pltpu.VMEM((CHUNK, D), jnp.float32),
            pltpu.VMEM((CHUNK, D), jnp.float32),
            pltpu.VMEM_SHARED((N_PAD, D), jnp.float32),
        ],
    )(phi, idx3)


MB = 400


def _mm_body(p_ref, wa_ref, wv_ref, wd_ref, ba_ref, oa_ref, ov_ref, od_ref):
    a = p_ref[0] + p_ref[1]
    oa_ref[...] = (jnp.dot(a, wa_ref[...], preferred_element_type=jnp.float32)
                   + ba_ref[...])
    ov_ref[...] = jnp.dot(a, wv_ref[...], preferred_element_type=jnp.float32)
    od_ref[...] = jnp.dot(a, wd_ref[...], preferred_element_type=jnp.float32)


def _node_matmuls(p3, wa_b, wv_b, wd_b, ba2):
    full = lambda i: (0, 0)
    return pl.pallas_call(
        _mm_body,
        grid=(N_NODES // MB,),
        in_specs=[
            pl.BlockSpec((2, MB, D), lambda i: (0, i, 0)),
            pl.BlockSpec((D, DIM_A), full),
            pl.BlockSpec((D, 3 * DIM_V), full),
            pl.BlockSpec((D, 9 * DIM_D), full),
            pl.BlockSpec((1, DIM_A), full),
        ],
        out_specs=[
            pl.BlockSpec((MB, DIM_A), lambda i: (i, 0)),
            pl.BlockSpec((MB, 3 * DIM_V), lambda i: (i, 0)),
            pl.BlockSpec((MB, 9 * DIM_D), lambda i: (i, 0)),
        ],
        out_shape=[
            jax.ShapeDtypeStruct((N_NODES, DIM_A), jnp.float32),
            jax.ShapeDtypeStruct((N_NODES, 3 * DIM_V), jnp.float32),
            jax.ShapeDtypeStruct((N_NODES, 9 * DIM_D), jnp.float32),
        ],
    )(p3, wa_b, wv_b, wd_b, ba2)


def _pad_col(col):
    return jnp.zeros((E_PAD, 1), jnp.float32).at[:N_EDGES, 0].set(col)


def kernel(r_ij, edge_src, W_a, b_a, W_v, W_d):
    rx = _pad_col(r_ij[:, 0])
    ry = _pad_col(r_ij[:, 1])
    rz = _pad_col(r_ij[:, 2])
    idx_pad = jnp.concatenate([
        edge_src.astype(jnp.int32),
        jnp.full((E_PAD - N_EDGES,), N_NODES, jnp.int32),
    ]).reshape(NW, CH_PER_W, CHUNK)

    phi = _features(rx, ry, rz)
    partials = _scatter_partials(phi, idx_pad).reshape(2, N_PAD, D)

    wa_b = jnp.zeros((D, DIM_A), jnp.float32).at[0:N_RAD].set(W_a)
    wv_b = jnp.zeros((D, 3 * DIM_V), jnp.float32)
    wd_b = jnp.zeros((D, 9 * DIM_D), jnp.float32)
    for x in range(3):
        wv_b = wv_b.at[8 * (1 + x):8 * (2 + x), x::3].set(W_v)
        for y in range(3):
            t = 4 + 3 * x + y
            g = 3 * x + y
            wd_b = wd_b.at[8 * t:8 * (t + 1), g::9].set(W_d)

    oa, ov, od = _node_matmuls(partials, wa_b, wv_b, wd_b,
                               b_a.reshape(1, DIM_A))
    B_a = oa
    B_v = ov.reshape(N_NODES, DIM_V, 3)
    B_d = od.reshape(N_NODES, DIM_D, 3, 3)
    return (B_a, B_v, B_d)


# speedup vs baseline: 31.4863x; 1.0004x over previous
import jax
import jax.numpy as jnp
import numpy as np
from jax import lax
from jax.experimental import pallas as pl
from jax.experimental.pallas import tpu as pltpu
from jax.experimental.pallas import tpu_sc as plsc

R0 = 6.0
N_NODES = 10000
N_EDGES = 640000
N_RAD = 8
DIM_A, DIM_V, DIM_D = 128, 64, 32

D = 128
NW = 32
CHUNK = 128
CH_PER_W = 160
IDXG = 16
NGRP = CH_PER_W // IDXG
EDGES_PW = CHUNK * CH_PER_W
E_PAD = NW * EDGES_PW
N_TILE_ROWS = 632
N_PAD = 16 * N_TILE_ROWS

FB = 2048

_A_SEL = [0, 1, 2, 3, 1, 1, 1, 2, 2, 2, 3, 3, 3, 0, 0, 0]
_B_SEL = [0, 0, 0, 0, 1, 2, 3, 1, 2, 3, 1, 2, 3, 0, 0, 0]


def _feat_body(rx_ref, ry_ref, rz_ref, phi_ref):
    rx, ry, rz = rx_ref[...], ry_ref[...], rz_ref[...]
    r_sq = rx * rx + ry * ry + rz * rz
    x_sq = r_sq * (1.0 / R0)
    w = jnp.maximum(1.0 - x_sq, 0.0)
    s = jnp.sqrt(x_sq)
    cc = 17.0 / R0
    norm = jnp.sqrt(r_sq * (cc * cc) + 1e-12)
    scale = (jnp.tanh(norm) / norm) * cc
    rvx, rvy, rvz = rx * scale, ry * scale, rz * scale

    f = lax.broadcasted_iota(jnp.int32, (1, D), 1)
    c_lane = (f & 7).astype(jnp.float32) * np.pi
    t_lane = f >> 3
    rad = jnp.cos(c_lane * s) * w

    def sel(code_per_t):
        code = jnp.zeros((1, D), jnp.int32)
        for t, v in enumerate(code_per_t):
            if v:
                code = jnp.where(t_lane == t, v, code)
        out = jnp.ones_like(rad)
        out = jnp.where(code == 1, rvx, out)
        out = jnp.where(code == 2, rvy, out)
        out = jnp.where(code == 3, rvz, out)
        return out

    m = sel(_A_SEL) * sel(_B_SEL)
    valid = (t_lane < 13).astype(jnp.float32)
    phi_ref[...] = rad * m * valid


def _features(rx, ry, rz):
    col = pl.BlockSpec((FB, 1), lambda i: (i, 0))
    return pl.pallas_call(
        _feat_body,
        grid=(E_PAD // FB,),
        in_specs=[col, col, col],
        out_specs=pl.BlockSpec((FB, D), lambda i: (i, 0)),
        out_shape=jax.ShapeDtypeStruct((E_PAD, D), jnp.float32),
    )(rx, ry, rz)


def _scatter_body(phi_hbm, idx_hbm, out_hbm, idx_v, buf_a, buf_b, acc_sh):
    c = lax.axis_index("c")
    s = lax.axis_index("s")
    wid = s * 2 + c
    base = s * N_TILE_ROWS

    @pl.loop(0, CHUNK)
    def _zero_rows(i):
        @pl.loop(0, D // 16)
        def _zero_cols(j):
            buf_a[i, pl.ds(j * 16, 16)] = jnp.zeros((16,), jnp.float32)

    for t in range(4):
        pltpu.sync_copy(buf_a, acc_sh.at[pl.ds(base + t * CHUNK, CHUNK)])
    pltpu.sync_copy(buf_a.at[pl.ds(0, N_TILE_ROWS - 4 * CHUNK)],
                    acc_sh.at[pl.ds(base + 4 * CHUNK, N_TILE_ROWS - 4 * CHUNK)])
    plsc.subcore_barrier()

    ebase = wid * EDGES_PW
    for g in range(NGRP):
        pltpu.sync_copy(idx_hbm.at[wid, pl.ds(g * IDXG, IDXG)], idx_v)

        @pl.loop(0, IDXG)
        def _edge_chunk(k):
            pltpu.sync_copy(
                phi_hbm.at[pl.ds(ebase + (g * IDXG + k) * CHUNK, CHUNK)],
                buf_a)
            pltpu.sync_copy(buf_a, acc_sh.at[idx_v.at[k]], add=True)

    plsc.subcore_barrier()

    obase = c * N_PAD + base
    for t in range(4):
        pltpu.sync_copy(acc_sh.at[pl.ds(base + t * CHUNK, CHUNK)], buf_b)
        pltpu.sync_copy(buf_b, out_hbm.at[pl.ds(obase + t * CHUNK, CHUNK)])
    tail = N_TILE_ROWS - 4 * CHUNK
    pltpu.sync_copy(acc_sh.at[pl.ds(base + 4 * CHUNK, tail)],
                    buf_b.at[pl.ds(0, tail)])
    pltpu.sync_copy(buf_b.at[pl.ds(0, tail)],
                    out_hbm.at[pl.ds(obase + 4 * CHUNK, tail)])


def _scatter_partials(phi, idx3):
    mesh = plsc.VectorSubcoreMesh(core_axis_name="c", subcore_axis_name="s")
    return pl.kernel(
        _scatter_body,
        out_type=jax.ShapeDtypeStruct((2 * N_PAD, D), jnp.float32),
        mesh=mesh,
        scratch_types=[
            pltpu.VMEM((IDXG, CHUNK), jnp.int32),
            pltpu.VMEM((CHUNK, D), jnp.float32),
            pltpu.VMEM((CHUNK, D), jnp.float32),
            pltpu.VMEM_SHARED((N_PAD, D), jnp.float32),
        ],
        compiler_params=pltpu.CompilerParams(use_tc_tiling_on_sc=True),
    )(phi, idx3)


MB = 400


def _mm_body(p_ref, wa_ref, wv_ref, wd_ref, ba_ref, oa_ref, ov_ref, od_ref):
    a = p_ref[0] + p_ref[1]
    oa_ref[...] = (jnp.dot(a, wa_ref[...], preferred_element_type=jnp.float32)
                   + ba_ref[...])
    ov_ref[...] = jnp.dot(a, wv_ref[...], preferred_element_type=jnp.float32)
    od_ref[...] = jnp.dot(a, wd_ref[...], preferred_element_type=jnp.float32)


def _node_matmuls(p3, wa_b, wv_b, wd_b, ba2):
    full = lambda i: (0, 0)
    return pl.pallas_call(
        _mm_body,
        grid=(N_NODES // MB,),
        in_specs=[
            pl.BlockSpec((2, MB, D), lambda i: (0, i, 0)),
            pl.BlockSpec((D, DIM_A), full),
            pl.BlockSpec((D, 3 * DIM_V), full),
            pl.BlockSpec((D, 9 * DIM_D), full),
            pl.BlockSpec((1, DIM_A), full),
        ],
        out_specs=[
            pl.BlockSpec((MB, DIM_A), lambda i: (i, 0)),
            pl.BlockSpec((MB, 3 * DIM_V), lambda i: (i, 0)),
            pl.BlockSpec((MB, 9 * DIM_D), lambda i: (i, 0)),
        ],
        out_shape=[
            jax.ShapeDtypeStruct((N_NODES, DIM_A), jnp.float32),
            jax.ShapeDtypeStruct((N_NODES, 3 * DIM_V), jnp.float32),
            jax.ShapeDtypeStruct((N_NODES, 9 * DIM_D), jnp.float32),
        ],
    )(p3, wa_b, wv_b, wd_b, ba2)


def _pad_col(col):
    return jnp.zeros((E_PAD, 1), jnp.float32).at[:N_EDGES, 0].set(col)


def kernel(r_ij, edge_src, W_a, b_a, W_v, W_d):
    rx = _pad_col(r_ij[:, 0])
    ry = _pad_col(r_ij[:, 1])
    rz = _pad_col(r_ij[:, 2])
    idx_pad = jnp.concatenate([
        edge_src.astype(jnp.int32),
        jnp.full((E_PAD - N_EDGES,), N_NODES, jnp.int32),
    ]).reshape(NW, CH_PER_W, CHUNK)

    phi = _features(rx, ry, rz)
    partials = _scatter_partials(phi, idx_pad).reshape(2, N_PAD, D)

    wa_b = jnp.zeros((D, DIM_A), jnp.float32).at[0:N_RAD].set(W_a)
    wv_b = jnp.zeros((D, 3 * DIM_V), jnp.float32)
    wd_b = jnp.zeros((D, 9 * DIM_D), jnp.float32)
    for x in range(3):
        wv_b = wv_b.at[8 * (1 + x):8 * (2 + x), x::3].set(W_v)
        for y in range(3):
            t = 4 + 3 * x + y
            g = 3 * x + y
            wd_b = wd_b.at[8 * t:8 * (t + 1), g::9].set(W_d)

    oa, ov, od = _node_matmuls(partials, wa_b, wv_b, wd_b,
                               b_a.reshape(1, DIM_A))
    B_a = oa
    B_v = ov.reshape(N_NODES, DIM_V, 3)
    B_d = od.reshape(N_NODES, DIM_D, 3, 3)
    return (B_a, B_v, B_d)


# speedup vs baseline: 84.5700x; 2.6859x over previous
import jax
import jax.numpy as jnp
import numpy as np
from jax import lax
from jax.experimental import pallas as pl
from jax.experimental.pallas import tpu as pltpu
from jax.experimental.pallas import tpu_sc as plsc

R0 = 6.0
N_NODES = 10000
N_EDGES = 640000
N_RAD = 8
DIM_A, DIM_V, DIM_D = 128, 64, 32

D = 128
NW = 32
CHUNK = 128
CH_PER_W = 160
IDXG = 16
NGRP = CH_PER_W // IDXG
EDGES_PW = CHUNK * CH_PER_W
E_PAD = NW * EDGES_PW
N_TILE_ROWS = 632
N_PAD = 16 * N_TILE_ROWS

FB = 2048

_A_SEL = [0, 1, 2, 3, 1, 1, 1, 2, 2, 2, 3, 3, 3, 0, 0, 0]
_B_SEL = [0, 0, 0, 0, 1, 2, 3, 1, 2, 3, 1, 2, 3, 0, 0, 0]


def _feat_body(rt_ref, phi_ref):
    rx = rt_ref[0:1, :]
    ry = rt_ref[1:2, :]
    rz = rt_ref[2:3, :]
    r_sq = rx * rx + ry * ry + rz * rz
    x_sq = r_sq * (1.0 / R0)
    w = jnp.maximum(1.0 - x_sq, 0.0)
    s = jnp.sqrt(x_sq)
    cc = 17.0 / R0
    norm = jnp.sqrt(r_sq * (cc * cc) + 1e-12)
    scale = (jnp.tanh(norm) / norm) * cc
    rvx, rvy, rvz = rx * scale, ry * scale, rz * scale

    f = lax.broadcasted_iota(jnp.int32, (D, 1), 0)
    c_sub = (f & 7).astype(jnp.float32) * np.pi
    t_sub = f >> 3
    radT = jnp.cos(c_sub * s) * w

    def sel(code_per_t):
        code = jnp.zeros((D, 1), jnp.int32)
        for t, v in enumerate(code_per_t):
            if v:
                code = jnp.where(t_sub == t, v, code)
        out = jnp.ones((D, FB), jnp.float32)
        out = jnp.where(code == 1, rvx, out)
        out = jnp.where(code == 2, rvy, out)
        out = jnp.where(code == 3, rvz, out)
        return out

    valid = (t_sub < 13).astype(jnp.float32)
    phi_t = radT * (sel(_A_SEL) * sel(_B_SEL)) * valid

    rr = lax.broadcasted_iota(jnp.int32, (D, D), 0)
    cc2 = lax.broadcasted_iota(jnp.int32, (D, D), 1)
    eye = (rr == cc2).astype(jnp.float32)
    phi_ref[...] = lax.dot_general(phi_t, eye, (((0,), (0,)), ((), ())),
                                   preferred_element_type=jnp.float32)


def _features(rt):
    return pl.pallas_call(
        _feat_body,
        grid=(E_PAD // FB,),
        in_specs=[pl.BlockSpec((8, FB), lambda i: (0, i))],
        out_specs=pl.BlockSpec((FB, D), lambda i: (i, 0)),
        out_shape=jax.ShapeDtypeStruct((E_PAD, D), jnp.float32),
    )(rt)


def _scatter_body(phi_hbm, idx_hbm, out_hbm, idx_v, buf_a, buf_b, acc_sh):
    c = lax.axis_index("c")
    s = lax.axis_index("s")
    wid = s * 2 + c
    base = s * N_TILE_ROWS

    @pl.loop(0, CHUNK)
    def _zero_rows(i):
        @pl.loop(0, D // 16)
        def _zero_cols(j):
            buf_a[i, pl.ds(j * 16, 16)] = jnp.zeros((16,), jnp.float32)

    for t in range(4):
        pltpu.sync_copy(buf_a, acc_sh.at[pl.ds(base + t * CHUNK, CHUNK)])
    pltpu.sync_copy(buf_a.at[pl.ds(0, N_TILE_ROWS - 4 * CHUNK)],
                    acc_sh.at[pl.ds(base + 4 * CHUNK, N_TILE_ROWS - 4 * CHUNK)])
    plsc.subcore_barrier()

    ebase = wid * EDGES_PW
    for g in range(NGRP):
        pltpu.sync_copy(idx_hbm.at[wid, pl.ds(g * IDXG, IDXG)], idx_v)

        @pl.loop(0, IDXG)
        def _edge_chunk(k):
            pltpu.sync_copy(
                phi_hbm.at[pl.ds(ebase + (g * IDXG + k) * CHUNK, CHUNK)],
                buf_a)
            pltpu.sync_copy(buf_a, acc_sh.at[idx_v.at[k]], add=True)

    plsc.subcore_barrier()

    obase = c * N_PAD + base
    for t in range(4):
        pltpu.sync_copy(acc_sh.at[pl.ds(base + t * CHUNK, CHUNK)], buf_b)
        pltpu.sync_copy(buf_b, out_hbm.at[pl.ds(obase + t * CHUNK, CHUNK)])
    tail = N_TILE_ROWS - 4 * CHUNK
    pltpu.sync_copy(acc_sh.at[pl.ds(base + 4 * CHUNK, tail)],
                    buf_b.at[pl.ds(0, tail)])
    pltpu.sync_copy(buf_b.at[pl.ds(0, tail)],
                    out_hbm.at[pl.ds(obase + 4 * CHUNK, tail)])


def _scatter_partials(phi, idx3):
    mesh = plsc.VectorSubcoreMesh(core_axis_name="c", subcore_axis_name="s")
    return pl.kernel(
        _scatter_body,
        out_type=jax.ShapeDtypeStruct((2 * N_PAD, D), jnp.float32),
        mesh=mesh,
        scratch_types=[
            pltpu.VMEM((IDXG, CHUNK), jnp.int32),
            pltpu.VMEM((CHUNK, D), jnp.float32),
            pltpu.VMEM((CHUNK, D), jnp.float32),
            pltpu.VMEM_SHARED((N_PAD, D), jnp.float32),
        ],
        compiler_params=pltpu.CompilerParams(use_tc_tiling_on_sc=True),
    )(phi, idx3)


MB = 400


def _mm_body(p_ref, wa_ref, wv_ref, wd_ref, ba_ref, oa_ref, ov_ref, od_ref):
    a = p_ref[0] + p_ref[1]
    oa_ref[...] = (jnp.dot(a, wa_ref[...], preferred_element_type=jnp.float32)
                   + ba_ref[...])
    ov_ref[...] = jnp.dot(a, wv_ref[...], preferred_element_type=jnp.float32)
    od_ref[...] = jnp.dot(a, wd_ref[...], preferred_element_type=jnp.float32)


def _node_matmuls(p3, wa_b, wv_b, wd_b, ba2):
    full = lambda i: (0, 0)
    return pl.pallas_call(
        _mm_body,
        grid=(N_NODES // MB,),
        in_specs=[
            pl.BlockSpec((2, MB, D), lambda i: (0, i, 0)),
            pl.BlockSpec((D, DIM_A), full),
            pl.BlockSpec((D, 3 * DIM_V), full),
            pl.BlockSpec((D, 9 * DIM_D), full),
            pl.BlockSpec((1, DIM_A), full),
        ],
        out_specs=[
            pl.BlockSpec((MB, DIM_A), lambda i: (i, 0)),
            pl.BlockSpec((MB, 3 * DIM_V), lambda i: (i, 0)),
            pl.BlockSpec((MB, 9 * DIM_D), lambda i: (i, 0)),
        ],
        out_shape=[
            jax.ShapeDtypeStruct((N_NODES, DIM_A), jnp.float32),
            jax.ShapeDtypeStruct((N_NODES, 3 * DIM_V), jnp.float32),
            jax.ShapeDtypeStruct((N_NODES, 9 * DIM_D), jnp.float32),
        ],
    )(p3, wa_b, wv_b, wd_b, ba2)


def kernel(r_ij, edge_src, W_a, b_a, W_v, W_d):
    rt = jnp.zeros((8, E_PAD), jnp.float32).at[:3, :N_EDGES].set(r_ij.T)
    idx_pad = jnp.concatenate([
        edge_src.astype(jnp.int32),
        jnp.full((E_PAD - N_EDGES,), N_NODES, jnp.int32),
    ]).reshape(NW, CH_PER_W, CHUNK)

    phi = _features(rt)
    partials = _scatter_partials(phi, idx_pad).reshape(2, N_PAD, D)

    wa_b = jnp.zeros((D, DIM_A), jnp.float32).at[0:N_RAD].set(W_a)
    wv4 = jnp.zeros((D // 8, 8, DIM_V, 3), jnp.float32)
    wd4 = jnp.zeros((D // 8, 8, DIM_D, 9), jnp.float32)
    for x in range(3):
        wv4 = wv4.at[1 + x, :, :, x].set(W_v)
        for y in range(3):
            g = 3 * x + y
            wd4 = wd4.at[4 + g, :, :, g].set(W_d)
    wv_b = wv4.reshape(D, 3 * DIM_V)
    wd_b = wd4.reshape(D, 9 * DIM_D)

    oa, ov, od = _node_matmuls(partials, wa_b, wv_b, wd_b,
                               b_a.reshape(1, DIM_A))
    B_a = oa
    B_v = ov.reshape(N_NODES, DIM_V, 3)
    B_d = od.reshape(N_NODES, DIM_D, 3, 3)
    return (B_a, B_v, B_d)


# speedup vs baseline: 196.1942x; 2.3199x over previous
import jax
import jax.numpy as jnp
import numpy as np
from jax import lax
from jax.experimental import pallas as pl
from jax.experimental.pallas import tpu as pltpu
from jax.experimental.pallas import tpu_sc as plsc

R0 = 6.0
N_NODES = 10000
N_EDGES = 640000
N_RAD = 8
DIM_A, DIM_V, DIM_D = 128, 64, 32

D = 128
NW = 32
CHUNK = 128
CH_PER_W = 160
IDXG = 16
NGRP = CH_PER_W // IDXG
EDGES_PW = CHUNK * CH_PER_W
E_PAD = NW * EDGES_PW
N_TILE_ROWS = 632
N_PAD = 16 * N_TILE_ROWS

FB = 2048

_A_SEL = [0, 1, 2, 3, 1, 1, 1, 2, 2, 2, 3, 3, 3, 0, 0, 0]
_B_SEL = [0, 0, 0, 0, 1, 2, 3, 1, 2, 3, 1, 2, 3, 0, 0, 0]


def _feat_body(rt_ref, phi_ref):
    rx = rt_ref[0:1, :]
    ry = rt_ref[1:2, :]
    rz = rt_ref[2:3, :]
    r_sq = rx * rx + ry * ry + rz * rz
    x_sq = r_sq * (1.0 / R0)
    w = jnp.maximum(1.0 - x_sq, 0.0)
    s = jnp.sqrt(x_sq)
    cc = 17.0 / R0
    norm = jnp.sqrt(r_sq * (cc * cc) + 1e-12)
    scale = (jnp.tanh(norm) / norm) * cc
    rvx, rvy, rvz = rx * scale, ry * scale, rz * scale

    c1 = jnp.cos(np.pi * s)
    us = [jnp.ones_like(c1), c1]
    for _ in range(2, N_RAD):
        us.append(2.0 * c1 * us[-1] - us[-2])
    rad8 = jnp.concatenate([u * w for u in us], axis=0)

    zero = jnp.zeros_like(rx)
    mvec = jnp.concatenate([
        jnp.ones_like(rx), rvx, rvy, rvz,
        rvx * rvx, rvx * rvy, rvx * rvz,
        rvy * rvx, rvy * rvy, rvy * rvz,
        rvz * rvx, rvz * rvy, rvz * rvz,
        zero, zero, zero,
    ], axis=0)

    f = lax.broadcasted_iota(jnp.int32, (1, D), 1)
    pr = (lax.broadcasted_iota(jnp.int32, (N_RAD, D), 0)
          == (f & 7)).astype(jnp.float32)
    pm = (lax.broadcasted_iota(jnp.int32, (16, D), 0)
          == (f >> 3)).astype(jnp.float32)
    dn = (((0,), (0,)), ((), ()))
    rexp = lax.dot_general(rad8, pr, dn, preferred_element_type=jnp.float32)
    mexp = lax.dot_general(mvec, pm, dn, preferred_element_type=jnp.float32)
    phi_ref[...] = rexp * mexp


def _features(rt):
    return pl.pallas_call(
        _feat_body,
        grid=(E_PAD // FB,),
        in_specs=[pl.BlockSpec((8, FB), lambda i: (0, i))],
        out_specs=pl.BlockSpec((FB, D), lambda i: (i, 0)),
        out_shape=jax.ShapeDtypeStruct((E_PAD, D), jnp.float32),
    )(rt)


def _scatter_body(phi_hbm, idx_hbm, out_hbm, idx_v, buf_a, buf_b, acc_sh):
    c = lax.axis_index("c")
    s = lax.axis_index("s")
    wid = s * 2 + c
    base = s * N_TILE_ROWS

    @pl.loop(0, CHUNK)
    def _zero_rows(i):
        @pl.loop(0, D // 16)
        def _zero_cols(j):
            buf_a[i, pl.ds(j * 16, 16)] = jnp.zeros((16,), jnp.float32)

    for t in range(4):
        pltpu.sync_copy(buf_a, acc_sh.at[pl.ds(base + t * CHUNK, CHUNK)])
    pltpu.sync_copy(buf_a.at[pl.ds(0, N_TILE_ROWS - 4 * CHUNK)],
                    acc_sh.at[pl.ds(base + 4 * CHUNK, N_TILE_ROWS - 4 * CHUNK)])
    plsc.subcore_barrier()

    ebase = wid * EDGES_PW
    for g in range(NGRP):
        pltpu.sync_copy(idx_hbm.at[wid, pl.ds(g * IDXG, IDXG)], idx_v)

        @pl.loop(0, IDXG)
        def _edge_chunk(k):
            pltpu.sync_copy(
                phi_hbm.at[pl.ds(ebase + (g * IDXG + k) * CHUNK, CHUNK)],
                buf_a)
            pltpu.sync_copy(buf_a, acc_sh.at[idx_v.at[k]], add=True)

    plsc.subcore_barrier()

    obase = c * N_PAD + base
    for t in range(4):
        pltpu.sync_copy(acc_sh.at[pl.ds(base + t * CHUNK, CHUNK)], buf_b)
        pltpu.sync_copy(buf_b, out_hbm.at[pl.ds(obase + t * CHUNK, CHUNK)])
    tail = N_TILE_ROWS - 4 * CHUNK
    pltpu.sync_copy(acc_sh.at[pl.ds(base + 4 * CHUNK, tail)],
                    buf_b.at[pl.ds(0, tail)])
    pltpu.sync_copy(buf_b.at[pl.ds(0, tail)],
                    out_hbm.at[pl.ds(obase + 4 * CHUNK, tail)])


def _scatter_partials(phi, idx3):
    mesh = plsc.VectorSubcoreMesh(core_axis_name="c", subcore_axis_name="s")
    return pl.kernel(
        _scatter_body,
        out_type=jax.ShapeDtypeStruct((2 * N_PAD, D), jnp.float32),
        mesh=mesh,
        scratch_types=[
            pltpu.VMEM((IDXG, CHUNK), jnp.int32),
            pltpu.VMEM((CHUNK, D), jnp.float32),
            pltpu.VMEM((CHUNK, D), jnp.float32),
            pltpu.VMEM_SHARED((N_PAD, D), jnp.float32),
        ],
        compiler_params=pltpu.CompilerParams(use_tc_tiling_on_sc=True),
    )(phi, idx3)


MB = 400


def _mm_body(p_ref, wa_ref, wv_ref, wd_ref, ba_ref, oa_ref, ov_ref, od_ref):
    a = p_ref[0] + p_ref[1]
    oa_ref[...] = (jnp.dot(a, wa_ref[...], preferred_element_type=jnp.float32)
                   + ba_ref[...])
    ov_ref[...] = jnp.dot(a, wv_ref[...], preferred_element_type=jnp.float32)
    od_ref[...] = jnp.dot(a, wd_ref[...], preferred_element_type=jnp.float32)


def _node_matmuls(p3, wa_b, wv_b, wd_b, ba2):
    full = lambda i: (0, 0)
    return pl.pallas_call(
        _mm_body,
        grid=(N_NODES // MB,),
        in_specs=[
            pl.BlockSpec((2, MB, D), lambda i: (0, i, 0)),
            pl.BlockSpec((D, DIM_A), full),
            pl.BlockSpec((D, 3 * DIM_V), full),
            pl.BlockSpec((D, 9 * DIM_D), full),
            pl.BlockSpec((1, DIM_A), full),
        ],
        out_specs=[
            pl.BlockSpec((MB, DIM_A), lambda i: (i, 0)),
            pl.BlockSpec((MB, 3 * DIM_V), lambda i: (i, 0)),
            pl.BlockSpec((MB, 9 * DIM_D), lambda i: (i, 0)),
        ],
        out_shape=[
            jax.ShapeDtypeStruct((N_NODES, DIM_A), jnp.float32),
            jax.ShapeDtypeStruct((N_NODES, 3 * DIM_V), jnp.float32),
            jax.ShapeDtypeStruct((N_NODES, 9 * DIM_D), jnp.float32),
        ],
    )(p3, wa_b, wv_b, wd_b, ba2)


def kernel(r_ij, edge_src, W_a, b_a, W_v, W_d):
    rt = jnp.zeros((8, E_PAD), jnp.float32).at[:3, :N_EDGES].set(r_ij.T)
    idx_pad = jnp.concatenate([
        edge_src.astype(jnp.int32),
        jnp.full((E_PAD - N_EDGES,), N_NODES, jnp.int32),
    ]).reshape(NW, CH_PER_W, CHUNK)

    phi = _features(rt)
    partials = _scatter_partials(phi, idx_pad).reshape(2, N_PAD, D)

    wa_b = jnp.zeros((D, DIM_A), jnp.float32).at[0:N_RAD].set(W_a)
    wv4 = jnp.zeros((D // 8, 8, DIM_V, 3), jnp.float32)
    wd4 = jnp.zeros((D // 8, 8, DIM_D, 9), jnp.float32)
    for x in range(3):
        wv4 = wv4.at[1 + x, :, :, x].set(W_v)
        for y in range(3):
            g = 3 * x + y
            wd4 = wd4.at[4 + g, :, :, g].set(W_d)
    wv_b = wv4.reshape(D, 3 * DIM_V)
    wd_b = wd4.reshape(D, 9 * DIM_D)

    oa, ov, od = _node_matmuls(partials, wa_b, wv_b, wd_b,
                               b_a.reshape(1, DIM_A))
    B_a = oa
    B_v = ov.reshape(N_NODES, DIM_V, 3)
    B_d = od.reshape(N_NODES, DIM_D, 3, 3)
    return (B_a, B_v, B_d)


# speedup vs baseline: 221.7661x; 1.1303x over previous
import jax
import jax.numpy as jnp
import numpy as np
from jax import lax
from jax.experimental import pallas as pl
from jax.experimental.pallas import tpu as pltpu
from jax.experimental.pallas import tpu_sc as plsc

R0 = 6.0
N_NODES = 10000
N_EDGES = 640000
N_RAD = 8
DIM_A, DIM_V, DIM_D = 128, 64, 32

D = 128
NW = 32
CHUNK = 128
CH_PER_W = 160
IDXG = 16
NGRP = CH_PER_W // IDXG
EDGES_PW = CHUNK * CH_PER_W
E_PAD = NW * EDGES_PW
N_TILE_ROWS = 632
N_PAD = 16 * N_TILE_ROWS

FB = 2048

_A_SEL = [0, 1, 2, 3, 1, 1, 1, 2, 2, 2, 3, 3, 3, 0, 0, 0]
_B_SEL = [0, 0, 0, 0, 1, 2, 3, 1, 2, 3, 1, 2, 3, 0, 0, 0]


def _feat_body(rt_ref, phi_ref):
    rx = rt_ref[0:1, :]
    ry = rt_ref[1:2, :]
    rz = rt_ref[2:3, :]
    r_sq = rx * rx + ry * ry + rz * rz
    x_sq = r_sq * (1.0 / R0)
    w = jnp.maximum(1.0 - x_sq, 0.0)
    s = jnp.sqrt(x_sq)
    cc = 17.0 / R0
    norm = jnp.sqrt(r_sq * (cc * cc) + 1e-12)
    scale = (jnp.tanh(norm) / norm) * cc
    rvx, rvy, rvz = rx * scale, ry * scale, rz * scale

    c1 = jnp.cos(np.pi * s)
    us = [jnp.ones_like(c1), c1]
    for _ in range(2, N_RAD):
        us.append(2.0 * c1 * us[-1] - us[-2])
    rad8 = jnp.concatenate([u * w for u in us], axis=0)

    zero = jnp.zeros_like(rx)
    mvec = jnp.concatenate([
        jnp.ones_like(rx), rvx, rvy, rvz,
        rvx * rvx, rvx * rvy, rvx * rvz,
        rvy * rvx, rvy * rvy, rvy * rvz,
        rvz * rvx, rvz * rvy, rvz * rvz,
        zero, zero, zero,
    ], axis=0)

    f = lax.broadcasted_iota(jnp.int32, (1, D), 1)
    pr = (lax.broadcasted_iota(jnp.int32, (N_RAD, D), 0)
          == (f & 7)).astype(jnp.float32)
    pm = (lax.broadcasted_iota(jnp.int32, (16, D), 0)
          == (f >> 3)).astype(jnp.float32)
    dn = (((0,), (0,)), ((), ()))
    rexp = lax.dot_general(rad8, pr, dn, preferred_element_type=jnp.float32)
    mexp = lax.dot_general(mvec, pm, dn, preferred_element_type=jnp.float32)
    phi_ref[...] = rexp * mexp


def _features(rt):
    return pl.pallas_call(
        _feat_body,
        grid=(E_PAD // FB,),
        in_specs=[pl.BlockSpec((8, FB), lambda i: (0, i))],
        out_specs=pl.BlockSpec((FB, D), lambda i: (i, 0)),
        out_shape=jax.ShapeDtypeStruct((E_PAD, D), jnp.float32),
    )(rt)


def _scatter_body(phi_hbm, idx_hbm, out_hbm, idx_v, buf_a, buf_b, acc_sh,
                  sem_a, sem_b):
    c = lax.axis_index("c")
    s = lax.axis_index("s")
    wid = s * 2 + c
    base = s * N_TILE_ROWS

    @pl.loop(0, CHUNK)
    def _zero_rows(i):
        @pl.loop(0, D // 16)
        def _zero_cols(j):
            buf_a[i, pl.ds(j * 16, 16)] = jnp.zeros((16,), jnp.float32)

    for t in range(4):
        pltpu.sync_copy(buf_a, acc_sh.at[pl.ds(base + t * CHUNK, CHUNK)])
    pltpu.sync_copy(buf_a.at[pl.ds(0, N_TILE_ROWS - 4 * CHUNK)],
                    acc_sh.at[pl.ds(base + 4 * CHUNK, N_TILE_ROWS - 4 * CHUNK)])
    plsc.subcore_barrier()

    ebase = wid * EDGES_PW

    def chunk_src(k):
        return phi_hbm.at[pl.ds(ebase + k * CHUNK, CHUNK)]

    for g in range(NGRP):
        pltpu.sync_copy(idx_hbm.at[wid, pl.ds(g * IDXG, IDXG)], idx_v)
        pltpu.async_copy(chunk_src(g * IDXG), buf_a, sem_a)

        @pl.loop(0, IDXG, step=2)
        def _edge_chunk(j):
            k0 = g * IDXG + j
            pltpu.make_async_copy(chunk_src(k0), buf_a, sem_a).wait()
            pltpu.async_copy(chunk_src(k0 + 1), buf_b, sem_b)
            pltpu.sync_copy(buf_a, acc_sh.at[idx_v.at[j]], add=True)
            pltpu.make_async_copy(chunk_src(k0 + 1), buf_b, sem_b).wait()

            @pl.when(j + 2 < IDXG)
            def _prefetch():
                pltpu.async_copy(chunk_src(k0 + 2), buf_a, sem_a)

            pltpu.sync_copy(buf_b, acc_sh.at[idx_v.at[j + 1]], add=True)

    plsc.subcore_barrier()

    obase = c * N_PAD + base
    for t in range(4):
        pltpu.sync_copy(acc_sh.at[pl.ds(base + t * CHUNK, CHUNK)], buf_b)
        pltpu.sync_copy(buf_b, out_hbm.at[pl.ds(obase + t * CHUNK, CHUNK)])
    tail = N_TILE_ROWS - 4 * CHUNK
    pltpu.sync_copy(acc_sh.at[pl.ds(base + 4 * CHUNK, tail)],
                    buf_b.at[pl.ds(0, tail)])
    pltpu.sync_copy(buf_b.at[pl.ds(0, tail)],
                    out_hbm.at[pl.ds(obase + 4 * CHUNK, tail)])


def _scatter_partials(phi, idx3):
    mesh = plsc.VectorSubcoreMesh(core_axis_name="c", subcore_axis_name="s")
    return pl.kernel(
        _scatter_body,
        out_type=jax.ShapeDtypeStruct((2 * N_PAD, D), jnp.float32),
        mesh=mesh,
        scratch_types=[
            pltpu.VMEM((IDXG, CHUNK), jnp.int32),
            pltpu.VMEM((CHUNK, D), jnp.float32),
            pltpu.VMEM((CHUNK, D), jnp.float32),
            pltpu.VMEM_SHARED((N_PAD, D), jnp.float32),
            pltpu.SemaphoreType.DMA,
            pltpu.SemaphoreType.DMA,
        ],
        compiler_params=pltpu.CompilerParams(use_tc_tiling_on_sc=True),
    )(phi, idx3)


MB = 400


def _mm_body(p_ref, wa_ref, wv_ref, wd_ref, ba_ref, oa_ref, ov_ref, od_ref):
    a = p_ref[0] + p_ref[1]
    oa_ref[...] = (jnp.dot(a, wa_ref[...], preferred_element_type=jnp.float32)
                   + ba_ref[...])
    ov_ref[...] = jnp.dot(a, wv_ref[...], preferred_element_type=jnp.float32)
    od_ref[...] = jnp.dot(a, wd_ref[...], preferred_element_type=jnp.float32)


def _node_matmuls(p3, wa_b, wv_b, wd_b, ba2):
    full = lambda i: (0, 0)
    return pl.pallas_call(
        _mm_body,
        grid=(N_NODES // MB,),
        in_specs=[
            pl.BlockSpec((2, MB, D), lambda i: (0, i, 0)),
            pl.BlockSpec((D, DIM_A), full),
            pl.BlockSpec((D, 3 * DIM_V), full),
            pl.BlockSpec((D, 9 * DIM_D), full),
            pl.BlockSpec((1, DIM_A), full),
        ],
        out_specs=[
            pl.BlockSpec((MB, DIM_A), lambda i: (i, 0)),
            pl.BlockSpec((MB, 3 * DIM_V), lambda i: (i, 0)),
            pl.BlockSpec((MB, 9 * DIM_D), lambda i: (i, 0)),
        ],
        out_shape=[
            jax.ShapeDtypeStruct((N_NODES, DIM_A), jnp.float32),
            jax.ShapeDtypeStruct((N_NODES, 3 * DIM_V), jnp.float32),
            jax.ShapeDtypeStruct((N_NODES, 9 * DIM_D), jnp.float32),
        ],
    )(p3, wa_b, wv_b, wd_b, ba2)


def kernel(r_ij, edge_src, W_a, b_a, W_v, W_d):
    rt = jnp.zeros((8, E_PAD), jnp.float32).at[:3, :N_EDGES].set(r_ij.T)
    idx_pad = jnp.concatenate([
        edge_src.astype(jnp.int32),
        jnp.full((E_PAD - N_EDGES,), N_NODES, jnp.int32),
    ]).reshape(NW, CH_PER_W, CHUNK)

    phi = _features(rt)
    partials = _scatter_partials(phi, idx_pad).reshape(2, N_PAD, D)

    wa_b = jnp.zeros((D, DIM_A), jnp.float32).at[0:N_RAD].set(W_a)
    wv4 = jnp.zeros((D // 8, 8, DIM_V, 3), jnp.float32)
    wd4 = jnp.zeros((D // 8, 8, DIM_D, 9), jnp.float32)
    for x in range(3):
        wv4 = wv4.at[1 + x, :, :, x].set(W_v)
        for y in range(3):
            g = 3 * x + y
            wd4 = wd4.at[4 + g, :, :, g].set(W_d)
    wv_b = wv4.reshape(D, 3 * DIM_V)
    wd_b = wd4.reshape(D, 9 * DIM_D)

    oa, ov, od = _node_matmuls(partials, wa_b, wv_b, wd_b,
                               b_a.reshape(1, DIM_A))
    B_a = oa
    B_v = ov.reshape(N_NODES, DIM_V, 3)
    B_d = od.reshape(N_NODES, DIM_D, 3, 3)
    return (B_a, B_v, B_d)


# speedup vs baseline: 260.2466x; 1.1735x over previous
import jax
import jax.numpy as jnp
import numpy as np
from jax import lax
from jax.experimental import pallas as pl
from jax.experimental.pallas import tpu as pltpu
from jax.experimental.pallas import tpu_sc as plsc

R0 = 6.0
N_NODES = 10000
N_EDGES = 640000
N_RAD = 8
DIM_A, DIM_V, DIM_D = 128, 64, 32

D = 128
NW = 32
CHUNK = 128
CH_PER_W = 80
IDXG = 16
NGRP = CH_PER_W // IDXG
EDGES_PW = CHUNK * CH_PER_W
E_HALF = NW * EDGES_PW
E_PAD = 2 * E_HALF
N_TILE_ROWS = 632
N_PAD = 16 * N_TILE_ROWS

FB = 2048

_A_SEL = [0, 1, 2, 3, 1, 1, 1, 2, 2, 2, 3, 3, 3, 0, 0, 0]
_B_SEL = [0, 0, 0, 0, 1, 2, 3, 1, 2, 3, 1, 2, 3, 0, 0, 0]


def _feat_body(rt_ref, phi_ref):
    rx = rt_ref[0:1, :]
    ry = rt_ref[1:2, :]
    rz = rt_ref[2:3, :]
    r_sq = rx * rx + ry * ry + rz * rz
    x_sq = r_sq * (1.0 / R0)
    w = jnp.maximum(1.0 - x_sq, 0.0)
    s = jnp.sqrt(x_sq)
    cc = 17.0 / R0
    norm = jnp.sqrt(r_sq * (cc * cc) + 1e-12)
    scale = (jnp.tanh(norm) / norm) * cc
    rvx, rvy, rvz = rx * scale, ry * scale, rz * scale

    c1 = jnp.cos(np.pi * s)
    us = [jnp.ones_like(c1), c1]
    for _ in range(2, N_RAD):
        us.append(2.0 * c1 * us[-1] - us[-2])
    rad8 = jnp.concatenate([u * w for u in us], axis=0)

    zero = jnp.zeros_like(rx)
    mvec = jnp.concatenate([
        jnp.ones_like(rx), rvx, rvy, rvz,
        rvx * rvx, rvx * rvy, rvx * rvz,
        rvy * rvx, rvy * rvy, rvy * rvz,
        rvz * rvx, rvz * rvy, rvz * rvz,
        zero, zero, zero,
    ], axis=0)

    f = lax.broadcasted_iota(jnp.int32, (1, D), 1)
    pr = (lax.broadcasted_iota(jnp.int32, (N_RAD, D), 0)
          == (f & 7)).astype(jnp.float32)
    pm = (lax.broadcasted_iota(jnp.int32, (16, D), 0)
          == (f >> 3)).astype(jnp.float32)
    dn = (((0,), (0,)), ((), ()))
    rexp = lax.dot_general(rad8, pr, dn, preferred_element_type=jnp.float32)
    mexp = lax.dot_general(mvec, pm, dn, preferred_element_type=jnp.float32)
    phi_ref[...] = rexp * mexp


def _features(rt, half):
    off = half * (E_HALF // FB)
    return pl.pallas_call(
        _feat_body,
        grid=(E_HALF // FB,),
        in_specs=[pl.BlockSpec((8, FB), lambda i: (0, i + off))],
        out_specs=pl.BlockSpec((FB, D), lambda i: (i, 0)),
        out_shape=jax.ShapeDtypeStruct((E_HALF, D), jnp.float32),
    )(rt)


def _scatter_body(phi_hbm, idx_hbm, out_hbm, idx_v, buf_a, buf_b, acc_sh,
                  sem_a, sem_b):
    c = lax.axis_index("c")
    s = lax.axis_index("s")
    wid = s * 2 + c
    base = s * N_TILE_ROWS

    @pl.loop(0, CHUNK)
    def _zero_rows(i):
        @pl.loop(0, D // 16)
        def _zero_cols(j):
            buf_a[i, pl.ds(j * 16, 16)] = jnp.zeros((16,), jnp.float32)

    for t in range(4):
        pltpu.sync_copy(buf_a, acc_sh.at[pl.ds(base + t * CHUNK, CHUNK)])
    pltpu.sync_copy(buf_a.at[pl.ds(0, N_TILE_ROWS - 4 * CHUNK)],
                    acc_sh.at[pl.ds(base + 4 * CHUNK, N_TILE_ROWS - 4 * CHUNK)])
    plsc.subcore_barrier()

    ebase = wid * EDGES_PW

    def chunk_src(k):
        return phi_hbm.at[pl.ds(ebase + k * CHUNK, CHUNK)]

    for g in range(NGRP):
        pltpu.sync_copy(idx_hbm.at[wid, pl.ds(g * IDXG, IDXG)], idx_v)
        pltpu.async_copy(chunk_src(g * IDXG), buf_a, sem_a)

        @pl.loop(0, IDXG, step=2)
        def _edge_chunk(j):
            k0 = g * IDXG + j
            pltpu.make_async_copy(chunk_src(k0), buf_a, sem_a).wait()
            pltpu.async_copy(chunk_src(k0 + 1), buf_b, sem_b)
            pltpu.sync_copy(buf_a, acc_sh.at[idx_v.at[j]], add=True)
            pltpu.make_async_copy(chunk_src(k0 + 1), buf_b, sem_b).wait()

            @pl.when(j + 2 < IDXG)
            def _prefetch():
                pltpu.async_copy(chunk_src(k0 + 2), buf_a, sem_a)

            pltpu.sync_copy(buf_b, acc_sh.at[idx_v.at[j + 1]], add=True)

    plsc.subcore_barrier()

    obase = c * N_PAD + base
    for t in range(4):
        pltpu.sync_copy(acc_sh.at[pl.ds(base + t * CHUNK, CHUNK)], buf_b)
        pltpu.sync_copy(buf_b, out_hbm.at[pl.ds(obase + t * CHUNK, CHUNK)])
    tail = N_TILE_ROWS - 4 * CHUNK
    pltpu.sync_copy(acc_sh.at[pl.ds(base + 4 * CHUNK, tail)],
                    buf_b.at[pl.ds(0, tail)])
    pltpu.sync_copy(buf_b.at[pl.ds(0, tail)],
                    out_hbm.at[pl.ds(obase + 4 * CHUNK, tail)])


def _scatter_partials(phi, idx3):
    mesh = plsc.VectorSubcoreMesh(core_axis_name="c", subcore_axis_name="s")
    return pl.kernel(
        _scatter_body,
        out_type=jax.ShapeDtypeStruct((2 * N_PAD, D), jnp.float32),
        mesh=mesh,
        scratch_types=[
            pltpu.VMEM((IDXG, CHUNK), jnp.int32),
            pltpu.VMEM((CHUNK, D), jnp.float32),
            pltpu.VMEM((CHUNK, D), jnp.float32),
            pltpu.VMEM_SHARED((N_PAD, D), jnp.float32),
            pltpu.SemaphoreType.DMA,
            pltpu.SemaphoreType.DMA,
        ],
        compiler_params=pltpu.CompilerParams(use_tc_tiling_on_sc=True),
    )(phi, idx3)


MB = 400


def _mm_body(p_ref, q_ref, wa_ref, wv_ref, wd_ref, ba_ref,
             oa_ref, ov_ref, od_ref):
    a = p_ref[0] + p_ref[1] + q_ref[0] + q_ref[1]
    oa_ref[...] = (jnp.dot(a, wa_ref[...], preferred_element_type=jnp.float32)
                   + ba_ref[...])
    ov_ref[...] = jnp.dot(a, wv_ref[...], preferred_element_type=jnp.float32)
    od_ref[...] = jnp.dot(a, wd_ref[...], preferred_element_type=jnp.float32)


def _node_matmuls(p3, q3, wa_b, wv_b, wd_b, ba2):
    full = lambda i: (0, 0)
    return pl.pallas_call(
        _mm_body,
        grid=(N_NODES // MB,),
        in_specs=[
            pl.BlockSpec((2, MB, D), lambda i: (0, i, 0)),
            pl.BlockSpec((2, MB, D), lambda i: (0, i, 0)),
            pl.BlockSpec((D, DIM_A), full),
            pl.BlockSpec((D, 3 * DIM_V), full),
            pl.BlockSpec((D, 9 * DIM_D), full),
            pl.BlockSpec((1, DIM_A), full),
        ],
        out_specs=[
            pl.BlockSpec((MB, DIM_A), lambda i: (i, 0)),
            pl.BlockSpec((MB, 3 * DIM_V), lambda i: (i, 0)),
            pl.BlockSpec((MB, 9 * DIM_D), lambda i: (i, 0)),
        ],
        out_shape=[
            jax.ShapeDtypeStruct((N_NODES, DIM_A), jnp.float32),
            jax.ShapeDtypeStruct((N_NODES, 3 * DIM_V), jnp.float32),
            jax.ShapeDtypeStruct((N_NODES, 9 * DIM_D), jnp.float32),
        ],
    )(p3, q3, wa_b, wv_b, wd_b, ba2)


def kernel(r_ij, edge_src, W_a, b_a, W_v, W_d):
    rt = jnp.zeros((8, E_PAD), jnp.float32).at[:3, :N_EDGES].set(r_ij.T)
    idx_pad = jnp.concatenate([
        edge_src.astype(jnp.int32),
        jnp.full((E_PAD - N_EDGES,), N_NODES, jnp.int32),
    ]).reshape(2, NW, CH_PER_W, CHUNK)

    phi_a = _features(rt, 0)
    part_a = _scatter_partials(phi_a, idx_pad[0]).reshape(2, N_PAD, D)
    phi_b = _features(rt, 1)
    part_b = _scatter_partials(phi_b, idx_pad[1]).reshape(2, N_PAD, D)

    wa_b = jnp.zeros((D, DIM_A), jnp.float32).at[0:N_RAD].set(W_a)
    wv4 = jnp.zeros((D // 8, 8, DIM_V, 3), jnp.float32)
    wd4 = jnp.zeros((D // 8, 8, DIM_D, 9), jnp.float32)
    for x in range(3):
        wv4 = wv4.at[1 + x, :, :, x].set(W_v)
        for y in range(3):
            g = 3 * x + y
            wd4 = wd4.at[4 + g, :, :, g].set(W_d)
    wv_b = wv4.reshape(D, 3 * DIM_V)
    wd_b = wd4.reshape(D, 9 * DIM_D)

    oa, ov, od = _node_matmuls(part_a, part_b, wa_b, wv_b, wd_b,
                               b_a.reshape(1, DIM_A))
    B_a = oa
    B_v = ov.reshape(N_NODES, DIM_V, 3)
    B_d = od.reshape(N_NODES, DIM_D, 3, 3)
    return (B_a, B_v, B_d)


# speedup vs baseline: 304.8801x; 1.1715x over previous
import jax
import jax.numpy as jnp
import numpy as np
from jax import lax
from jax.experimental import pallas as pl
from jax.experimental.pallas import tpu as pltpu
from jax.experimental.pallas import tpu_sc as plsc

R0 = 6.0
N_NODES = 10000
N_EDGES = 640000
N_RAD = 8
DIM_A, DIM_V, DIM_D = 128, 64, 32

D = 128
NW = 32
CHUNK = 128
CH_PER_W = 80
IDXG = 16
NGRP = CH_PER_W // IDXG
EDGES_PW = CHUNK * CH_PER_W
E_HALF = NW * EDGES_PW
E_PAD = 2 * E_HALF
N_TILE_ROWS = 632
N_PAD = 16 * N_TILE_ROWS

FB = 2048

_A_SEL = [0, 1, 2, 3, 1, 1, 1, 2, 2, 2, 3, 3, 3, 0, 0, 0]
_B_SEL = [0, 0, 0, 0, 1, 2, 3, 1, 2, 3, 1, 2, 3, 0, 0, 0]


def _feat_body(rt_ref, phi_ref):
    rx = rt_ref[0:1, :]
    ry = rt_ref[1:2, :]
    rz = rt_ref[2:3, :]
    r_sq = rx * rx + ry * ry + rz * rz
    x_sq = r_sq * (1.0 / R0)
    w = jnp.maximum(1.0 - x_sq, 0.0)
    s = jnp.sqrt(x_sq)
    cc = 17.0 / R0
    norm = jnp.sqrt(r_sq * (cc * cc) + 1e-12)
    scale = (jnp.tanh(norm) / norm) * cc
    rvx, rvy, rvz = rx * scale, ry * scale, rz * scale

    c1 = jnp.cos(np.pi * s)
    us = [jnp.ones_like(c1), c1]
    for _ in range(2, N_RAD):
        us.append(2.0 * c1 * us[-1] - us[-2])
    rad8 = jnp.concatenate([u * w for u in us], axis=0)

    zero = jnp.zeros_like(rx)
    mvec = jnp.concatenate([
        jnp.ones_like(rx), rvx, rvy, rvz,
        rvx * rvx, rvx * rvy, rvx * rvz,
        rvy * rvx, rvy * rvy, rvy * rvz,
        rvz * rvx, rvz * rvy, rvz * rvz,
        zero, zero, zero,
    ], axis=0)

    f = lax.broadcasted_iota(jnp.int32, (1, D), 1)
    pr = (lax.broadcasted_iota(jnp.int32, (N_RAD, D), 0)
          == (f & 7)).astype(jnp.float32)
    pm = (lax.broadcasted_iota(jnp.int32, (16, D), 0)
          == (f >> 3)).astype(jnp.float32)
    dn = (((0,), (0,)), ((), ()))
    rexp = lax.dot_general(rad8, pr, dn, preferred_element_type=jnp.float32)
    mexp = lax.dot_general(mvec, pm, dn, preferred_element_type=jnp.float32)
    phi_ref[...] = rexp * mexp


def _features(rt, half):
    off = half * (E_HALF // FB)
    return pl.pallas_call(
        _feat_body,
        grid=(E_HALF // FB,),
        in_specs=[pl.BlockSpec((8, FB), lambda i: (0, i + off))],
        out_specs=pl.BlockSpec((FB, D), lambda i: (i, 0)),
        out_shape=jax.ShapeDtypeStruct((E_HALF, D), jnp.float32),
    )(rt)


def _scatter_body(phi_hbm, idx_hbm, out_hbm, idx_v, buf_a, buf_b, acc_sh,
                  sem_a, sem_b):
    c = lax.axis_index("c")
    s = lax.axis_index("s")
    wid = s * 2 + c
    base = s * N_TILE_ROWS

    @pl.loop(0, CHUNK)
    def _zero_rows(i):
        @pl.loop(0, D // 16)
        def _zero_cols(j):
            buf_a[i, pl.ds(j * 16, 16)] = jnp.zeros((16,), jnp.float32)

    for t in range(4):
        pltpu.sync_copy(buf_a, acc_sh.at[pl.ds(base + t * CHUNK, CHUNK)])
    pltpu.sync_copy(buf_a.at[pl.ds(0, N_TILE_ROWS - 4 * CHUNK)],
                    acc_sh.at[pl.ds(base + 4 * CHUNK, N_TILE_ROWS - 4 * CHUNK)])
    plsc.subcore_barrier()

    ebase = wid * EDGES_PW

    def chunk_src(k):
        return phi_hbm.at[pl.ds(ebase + k * CHUNK, CHUNK)]

    for g in range(NGRP):
        pltpu.sync_copy(idx_hbm.at[wid, pl.ds(g * IDXG, IDXG)], idx_v)
        pltpu.async_copy(chunk_src(g * IDXG), buf_a, sem_a)

        @pl.loop(0, IDXG, step=2)
        def _edge_chunk(j):
            k0 = g * IDXG + j
            pltpu.make_async_copy(chunk_src(k0), buf_a, sem_a).wait()
            pltpu.async_copy(chunk_src(k0 + 1), buf_b, sem_b)
            pltpu.sync_copy(buf_a, acc_sh.at[idx_v.at[j]], add=True)
            pltpu.make_async_copy(chunk_src(k0 + 1), buf_b, sem_b).wait()

            @pl.when(j + 2 < IDXG)
            def _prefetch():
                pltpu.async_copy(chunk_src(k0 + 2), buf_a, sem_a)

            pltpu.sync_copy(buf_b, acc_sh.at[idx_v.at[j + 1]], add=True)

    plsc.subcore_barrier()

    obase = c * N_PAD + base
    for t in range(4):
        pltpu.sync_copy(acc_sh.at[pl.ds(base + t * CHUNK, CHUNK)], buf_b)
        pltpu.sync_copy(buf_b, out_hbm.at[pl.ds(obase + t * CHUNK, CHUNK)])
    tail = N_TILE_ROWS - 4 * CHUNK
    pltpu.sync_copy(acc_sh.at[pl.ds(base + 4 * CHUNK, tail)],
                    buf_b.at[pl.ds(0, tail)])
    pltpu.sync_copy(buf_b.at[pl.ds(0, tail)],
                    out_hbm.at[pl.ds(obase + 4 * CHUNK, tail)])


def _scatter_partials(phi, idx3):
    mesh = plsc.VectorSubcoreMesh(core_axis_name="c", subcore_axis_name="s")
    return pl.kernel(
        _scatter_body,
        out_type=jax.ShapeDtypeStruct((2 * N_PAD, D), jnp.float32),
        mesh=mesh,
        scratch_types=[
            pltpu.VMEM((IDXG, CHUNK), jnp.int32),
            pltpu.VMEM((CHUNK, D), jnp.float32),
            pltpu.VMEM((CHUNK, D), jnp.float32),
            pltpu.VMEM_SHARED((N_PAD, D), jnp.float32),
            pltpu.SemaphoreType.DMA,
            pltpu.SemaphoreType.DMA,
        ],
        compiler_params=pltpu.CompilerParams(use_tc_tiling_on_sc=True),
    )(phi, idx3)


def _mm_body(p_ref, q_ref, wa_ref, wv_ref, wd_ref, ba_ref,
             oa_ref, ov_ref, od_ref):
    a = p_ref[0] + p_ref[1] + q_ref[0] + q_ref[1]
    oa_ref[...] = (jnp.dot(a, wa_ref[...], preferred_element_type=jnp.float32)
                   + ba_ref[...])
    dn = (((0,), (1,)), ((), ()))
    ov_ref[...] = lax.dot_general(wv_ref[...], a, dn,
                                  preferred_element_type=jnp.float32)
    od_ref[...] = lax.dot_general(wd_ref[...], a, dn,
                                  preferred_element_type=jnp.float32)


def _node_matmuls(p3, q3, wa_b, wv_b, wd_b, ba2):
    full = lambda i: (0, 0)
    return pl.pallas_call(
        _mm_body,
        grid=(1,),
        in_specs=[
            pl.BlockSpec((2, N_NODES, D), lambda i: (0, 0, 0)),
            pl.BlockSpec((2, N_NODES, D), lambda i: (0, 0, 0)),
            pl.BlockSpec((D, DIM_A), full),
            pl.BlockSpec((D, 3 * DIM_V), full),
            pl.BlockSpec((D, 9 * DIM_D), full),
            pl.BlockSpec((1, DIM_A), full),
        ],
        out_specs=[
            pl.BlockSpec((N_NODES, DIM_A), full),
            pl.BlockSpec((3 * DIM_V, N_NODES), full),
            pl.BlockSpec((9 * DIM_D, N_NODES), full),
        ],
        out_shape=[
            jax.ShapeDtypeStruct((N_NODES, DIM_A), jnp.float32),
            jax.ShapeDtypeStruct((3 * DIM_V, N_NODES), jnp.float32),
            jax.ShapeDtypeStruct((9 * DIM_D, N_NODES), jnp.float32),
        ],
    )(p3, q3, wa_b, wv_b, wd_b, ba2)


def kernel(r_ij, edge_src, W_a, b_a, W_v, W_d):
    rt = jnp.zeros((8, E_PAD), jnp.float32).at[:3, :N_EDGES].set(r_ij.T)
    idx_pad = jnp.concatenate([
        edge_src.astype(jnp.int32),
        jnp.full((E_PAD - N_EDGES,), N_NODES, jnp.int32),
    ]).reshape(2, NW, CH_PER_W, CHUNK)

    phi_a = _features(rt, 0)
    part_a = _scatter_partials(phi_a, idx_pad[0]).reshape(2, N_PAD, D)
    phi_b = _features(rt, 1)
    part_b = _scatter_partials(phi_b, idx_pad[1]).reshape(2, N_PAD, D)

    wa_b = jnp.zeros((D, DIM_A), jnp.float32).at[0:N_RAD].set(W_a)
    wv4 = jnp.zeros((D // 8, 8, 3, DIM_V), jnp.float32)
    wd4 = jnp.zeros((D // 8, 8, 9, DIM_D), jnp.float32)
    for x in range(3):
        wv4 = wv4.at[1 + x, :, x, :].set(W_v)
        for y in range(3):
            g = 3 * x + y
            wd4 = wd4.at[4 + g, :, g, :].set(W_d)
    wv_b = wv4.reshape(D, 3 * DIM_V)
    wd_b = wd4.reshape(D, 9 * DIM_D)

    oa, ov, od = _node_matmuls(part_a, part_b, wa_b, wv_b, wd_b,
                               b_a.reshape(1, DIM_A))
    B_a = oa
    B_v = ov.reshape(3, DIM_V, N_NODES).transpose(2, 1, 0)
    B_d = od.reshape(3, 3, DIM_D, N_NODES).transpose(3, 2, 0, 1)
    return (B_a, B_v, B_d)
